# baseline (speedup 1.0000x reference)
import functools

import jax
import jax.numpy as jnp
from jax import lax
from jax.experimental import pallas as pl
from jax.experimental.pallas import tpu as pltpu
from jax.experimental.pallas import tpu_sc as plsc

HIDDEN = 1024
N_EXPERTS = 8
INTER = 1408
TOP_K = 2
T = 4096

TM = 256
P = T * TOP_K + N_EXPERTS * TM
NT = P // TM

NC = 2
NS = 16
NW = NC * NS


def _sc_dispatch(x, pos1r, pos2r, g1, g2, chunk, nbuf=3):
    D = x.shape[1]
    tpw = T // NW
    n_chunks = tpw // chunk
    mesh = plsc.VectorSubcoreMesh(core_axis_name="c", subcore_axis_name="s")

    @functools.partial(
        pl.kernel,
        mesh=mesh,
        out_type=(jax.ShapeDtypeStruct((P, D), x.dtype),
                  jax.ShapeDtypeStruct((P,), jnp.float32)),
        scratch_types=[
            pltpu.VMEM((n_chunks, chunk), jnp.int32),
            pltpu.VMEM((n_chunks, chunk), jnp.int32),
            pltpu.VMEM((tpw,), jnp.float32),
            pltpu.VMEM((tpw,), jnp.float32),
            [pltpu.VMEM((chunk, D), x.dtype) for _ in range(nbuf)],
            [pltpu.SemaphoreType.DMA for _ in range(nbuf)],
            [pltpu.SemaphoreType.DMA for _ in range(nbuf)],
            [pltpu.SemaphoreType.DMA for _ in range(nbuf)],
            pltpu.SemaphoreType.DMA,
        ],
    )
    def k(x_hbm, pos1_hbm, pos2_hbm, g1_hbm, g2_hbm, out_hbm, gate_hbm,
          p1_v, p2_v, g1_v, g2_v, bufs, isems, asems, bsems, gsem):
        wid = lax.axis_index("s") * NC + lax.axis_index("c")
        base = wid * tpw
        pltpu.sync_copy(pos1_hbm.at[pl.ds(wid * n_chunks, n_chunks)], p1_v)
        pltpu.sync_copy(pos2_hbm.at[pl.ds(wid * n_chunks, n_chunks)], p2_v)
        pltpu.sync_copy(g1_hbm.at[pl.ds(base, tpw)], g1_v)
        pltpu.sync_copy(g2_hbm.at[pl.ds(base, tpw)], g2_v)
        gd = []
        for c in range(n_chunks):
            gd.append(pltpu.async_copy(
                g1_v.at[pl.ds(c * chunk, chunk)],
                gate_hbm.at[p1_v.at[c]], gsem))
            gd.append(pltpu.async_copy(
                g2_v.at[pl.ds(c * chunk, chunk)],
                gate_hbm.at[p2_v.at[c]], gsem))
        in_d = [None] * n_chunks
        outA = [None] * n_chunks
        outB = [None] * n_chunks

        def flush(c):
            b = c % nbuf
            in_d[c].wait()
            outA[c] = pltpu.async_copy(
                bufs[b], out_hbm.at[p1_v.at[c]], asems[b])
            outB[c] = pltpu.async_copy(
                bufs[b], out_hbm.at[p2_v.at[c]], bsems[b])

        for c in range(n_chunks):
            b = c % nbuf
            if c >= nbuf:
                outA[c - nbuf].wait()
                outB[c - nbuf].wait()
            in_d[c] = pltpu.async_copy(
                x_hbm.at[pl.ds(base + c * chunk, chunk)], bufs[b], isems[b])
            if c >= 1:
                flush(c - 1)
        flush(n_chunks - 1)
        for c in range(max(0, n_chunks - nbuf), n_chunks):
            outA[c].wait()
            outB[c].wait()
        for d in gd:
            d.wait()

    return k(x, pos1r, pos2r, g1, g2)


def _sc_combine(yg, pos1, pos2, chunk, nbuf=2):
    D = yg.shape[1]
    lanes = 32 if yg.dtype == jnp.bfloat16 else 16
    tpw = T // NW
    n_chunks = tpw // chunk
    ncol = D // lanes
    mesh = plsc.VectorSubcoreMesh(core_axis_name="c", subcore_axis_name="s")

    @functools.partial(
        pl.kernel,
        mesh=mesh,
        out_type=jax.ShapeDtypeStruct((T, D), yg.dtype),
        scratch_types=[
            pltpu.VMEM((tpw,), jnp.int32),
            pltpu.VMEM((tpw,), jnp.int32),
            [pltpu.VMEM((chunk, D), yg.dtype) for _ in range(nbuf)],
            [pltpu.VMEM((chunk, D), yg.dtype) for _ in range(nbuf)],
            [pltpu.SemaphoreType.DMA for _ in range(nbuf)],
            [pltpu.SemaphoreType.DMA for _ in range(nbuf)],
            [pltpu.SemaphoreType.DMA for _ in range(nbuf)],
        ],
    )
    def k(yg_hbm, pos1_hbm, pos2_hbm, out_hbm,
          p1_v, p2_v, a_bufs, b_bufs, asems, bsems, osems):
        wid = lax.axis_index("s") * NC + lax.axis_index("c")
        base = wid * tpw
        pltpu.sync_copy(pos1_hbm.at[pl.ds(base, tpw)], p1_v)
        pltpu.sync_copy(pos2_hbm.at[pl.ds(base, tpw)], p2_v)
        inA = [None] * n_chunks
        inB = [None] * n_chunks
        out_d = [None] * n_chunks

        def process(c):
            s = c % nbuf
            inA[c].wait()
            inB[c].wait()
            a_v, b_v = a_bufs[s], b_bufs[s]

            def add_step(j, c2):
                r = j // ncol
                col = (j % ncol) * lanes
                a_v[r, pl.ds(col, lanes)] = (
                    a_v[r, pl.ds(col, lanes)] + b_v[r, pl.ds(col, lanes)]
                )
                return c2

            lax.fori_loop(0, chunk * ncol, add_step, 0, unroll=8)
            out_d[c] = pltpu.async_copy(
                a_v, out_hbm.at[pl.ds(base + c * chunk, chunk)], osems[s])

        for c in range(n_chunks):
            s = c % nbuf
            if c >= nbuf:
                out_d[c - nbuf].wait()
            inA[c] = pltpu.async_copy(
                yg_hbm.at[p1_v.at[pl.ds(c * chunk, chunk)]], a_bufs[s],
                asems[s])
            inB[c] = pltpu.async_copy(
                yg_hbm.at[p2_v.at[pl.ds(c * chunk, chunk)]], b_bufs[s],
                bsems[s])
            if c >= 1:
                process(c - 1)
        process(n_chunks - 1)
        for c in range(max(0, n_chunks - nbuf), n_chunks):
            out_d[c].wait()

    return k(yg, pos1, pos2)


NJ = 4
HJ = HIDDEN // NJ


def _tc_expert_matmul(xg, w1, w2, gates3, expert_map):
    def body(em_ref, xg_ref, w1_ref, w2_ref, g_ref, yg_ref, hs_ref):
        j = pl.program_id(1)

        @pl.when(j == 0)
        def _():
            xb = xg_ref[...].astype(jnp.bfloat16)
            h = lax.dot_general(
                xb, w1_ref[0].astype(jnp.bfloat16),
                (((1,), (1,)), ((), ())),
                preferred_element_type=jnp.float32,
            )
            h = h * jax.nn.sigmoid(h)
            hs_ref[...] = h.astype(jnp.bfloat16)

        y = lax.dot_general(
            hs_ref[...],
            w2_ref[0].astype(jnp.bfloat16),
            (((1,), (1,)), ((), ())),
            preferred_element_type=jnp.float32,
        )
        gv = g_ref[0, 0, :]
        yg_ref[...] = y * gv[:, None]

    grid_spec = pltpu.PrefetchScalarGridSpec(
        num_scalar_prefetch=1,
        grid=(NT, NJ),
        in_specs=[
            pl.BlockSpec((TM, HIDDEN), lambda i, j, em: (i, 0)),
            pl.BlockSpec((1, INTER, HIDDEN), lambda i, j, em: (em[i], 0, 0)),
            pl.BlockSpec((1, HJ, INTER), lambda i, j, em: (em[i], j, 0)),
            pl.BlockSpec((1, 1, TM), lambda i, j, em: (i, 0, 0)),
        ],
        out_specs=pl.BlockSpec((TM, HJ), lambda i, j, em: (i, j)),
        scratch_shapes=[pltpu.VMEM((TM, INTER), jnp.bfloat16)],
    )
    return pl.pallas_call(
        body,
        grid_spec=grid_spec,
        out_shape=jax.ShapeDtypeStruct((P, HIDDEN), jnp.float32),
        compiler_params=pltpu.CompilerParams(
            dimension_semantics=("arbitrary", "arbitrary"),
        ),
    )(expert_map, xg, w1, w2, gates3)


def kernel(x, router_logits, w1, w2):
    lg = router_logits.astype(jnp.float32)
    i1 = jnp.argmax(lg, axis=-1)
    eiota = jnp.arange(N_EXPERTS, dtype=jnp.int32)[None, :]
    oh1 = eiota == i1[:, None]
    lgm = jnp.where(oh1, -jnp.inf, lg)
    i2 = jnp.argmax(lgm, axis=-1)
    oh2 = eiota == i2[:, None]
    l1 = jnp.sum(jnp.where(oh1, lg, 0.0), axis=-1)
    l2 = jnp.sum(jnp.where(oh2, lg, 0.0), axis=-1)
    e2 = jnp.exp(l2 - l1)
    g1 = 1.0 / (1.0 + e2)
    g2 = e2 / (1.0 + e2)

    sel = oh1.astype(jnp.int32) + oh2.astype(jnp.int32)
    csum = jnp.cumsum(sel, axis=0)
    rank = csum - sel
    counts = csum[-1]
    padded = ((counts + TM - 1) // TM) * TM
    starts = jnp.concatenate(
        [jnp.zeros((1,), jnp.int32), jnp.cumsum(padded)[:-1].astype(jnp.int32)]
    )
    pos1 = jnp.sum(jnp.where(oh1, rank + starts[None, :], 0),
                   axis=-1).astype(jnp.int32)
    pos2 = jnp.sum(jnp.where(oh2, rank + starts[None, :], 0),
                   axis=-1).astype(jnp.int32)
    tile_starts = jnp.arange(NT, dtype=jnp.int32) * TM
    expert_map = (tile_starts[:, None] >= starts[None, 1:]).sum(
        axis=1).astype(jnp.int32)

    disp_chunk = 32
    nch = (T // NW) // disp_chunk
    pos1r = pos1.reshape(NW * nch, disp_chunk)
    pos2r = pos2.reshape(NW * nch, disp_chunk)
    xg, gatep = _sc_dispatch(x, pos1r, pos2r, g1, g2,
                             chunk=disp_chunk)

    gates3 = gatep.reshape(NT, 1, TM)
    yg = _tc_expert_matmul(xg, w1, w2, gates3, expert_map)

    out = _sc_combine(yg, pos1, pos2, chunk=16)
    return out.astype(x.dtype)


# speedup vs baseline: 1.6274x; 1.6274x over previous
import functools

import jax
import jax.numpy as jnp
from jax import lax
from jax.experimental import pallas as pl
from jax.experimental.pallas import tpu as pltpu
from jax.experimental.pallas import tpu_sc as plsc

HIDDEN = 1024
N_EXPERTS = 8
INTER = 1408
TOP_K = 2
T = 4096

TM = 256
P = T * TOP_K + N_EXPERTS * TM
NT = P // TM

NC = 2
NS = 16
NW = NC * NS


def _sc_dispatch(x, pos1r, pos2r, g1, g2, chunk, nbuf=3):
    D = x.shape[1]
    tpw = T // NW
    n_chunks = tpw // chunk
    mesh = plsc.VectorSubcoreMesh(core_axis_name="c", subcore_axis_name="s")

    @functools.partial(
        pl.kernel,
        mesh=mesh,
        out_type=(jax.ShapeDtypeStruct((P, D), x.dtype),
                  jax.ShapeDtypeStruct((P,), jnp.float32)),
        scratch_types=[
            pltpu.VMEM((n_chunks, chunk), jnp.int32),
            pltpu.VMEM((n_chunks, chunk), jnp.int32),
            pltpu.VMEM((tpw,), jnp.float32),
            pltpu.VMEM((tpw,), jnp.float32),
            [pltpu.VMEM((chunk, D), x.dtype) for _ in range(nbuf)],
            [pltpu.SemaphoreType.DMA for _ in range(nbuf)],
            [pltpu.SemaphoreType.DMA for _ in range(nbuf)],
            [pltpu.SemaphoreType.DMA for _ in range(nbuf)],
            pltpu.SemaphoreType.DMA,
        ],
    )
    def k(x_hbm, pos1_hbm, pos2_hbm, g1_hbm, g2_hbm, out_hbm, gate_hbm,
          p1_v, p2_v, g1_v, g2_v, bufs, isems, asems, bsems, gsem):
        wid = lax.axis_index("s") * NC + lax.axis_index("c")
        base = wid * tpw
        pltpu.sync_copy(pos1_hbm.at[pl.ds(wid * n_chunks, n_chunks)], p1_v)
        pltpu.sync_copy(pos2_hbm.at[pl.ds(wid * n_chunks, n_chunks)], p2_v)
        pltpu.sync_copy(g1_hbm.at[pl.ds(base, tpw)], g1_v)
        pltpu.sync_copy(g2_hbm.at[pl.ds(base, tpw)], g2_v)
        gd = []
        for c in range(n_chunks):
            gd.append(pltpu.async_copy(
                g1_v.at[pl.ds(c * chunk, chunk)],
                gate_hbm.at[p1_v.at[c]], gsem))
            gd.append(pltpu.async_copy(
                g2_v.at[pl.ds(c * chunk, chunk)],
                gate_hbm.at[p2_v.at[c]], gsem))
        in_d = [None] * n_chunks
        outA = [None] * n_chunks
        outB = [None] * n_chunks

        def flush(c):
            b = c % nbuf
            in_d[c].wait()
            outA[c] = pltpu.async_copy(
                bufs[b], out_hbm.at[p1_v.at[c]], asems[b])
            outB[c] = pltpu.async_copy(
                bufs[b], out_hbm.at[p2_v.at[c]], bsems[b])

        for c in range(n_chunks):
            b = c % nbuf
            if c >= nbuf:
                outA[c - nbuf].wait()
                outB[c - nbuf].wait()
            in_d[c] = pltpu.async_copy(
                x_hbm.at[pl.ds(base + c * chunk, chunk)], bufs[b], isems[b])
            if c >= 1:
                flush(c - 1)
        flush(n_chunks - 1)
        for c in range(max(0, n_chunks - nbuf), n_chunks):
            outA[c].wait()
            outB[c].wait()
        for d in gd:
            d.wait()

    return k(x, pos1r, pos2r, g1, g2)


def _sc_combine(yg, pos1, pos2, chunk, nbuf=2):
    D = yg.shape[1]
    lanes = 32 if yg.dtype == jnp.bfloat16 else 16
    tpw = T // NW
    n_chunks = tpw // chunk
    ncol = D // lanes
    mesh = plsc.VectorSubcoreMesh(core_axis_name="c", subcore_axis_name="s")

    @functools.partial(
        pl.kernel,
        mesh=mesh,
        out_type=jax.ShapeDtypeStruct((T, D), yg.dtype),
        scratch_types=[
            pltpu.VMEM((tpw,), jnp.int32),
            pltpu.VMEM((tpw,), jnp.int32),
            [pltpu.VMEM((chunk, D), yg.dtype) for _ in range(nbuf)],
            [pltpu.VMEM((chunk, D), yg.dtype) for _ in range(nbuf)],
            [pltpu.SemaphoreType.DMA for _ in range(nbuf)],
            [pltpu.SemaphoreType.DMA for _ in range(nbuf)],
            [pltpu.SemaphoreType.DMA for _ in range(nbuf)],
        ],
    )
    def k(yg_hbm, pos1_hbm, pos2_hbm, out_hbm,
          p1_v, p2_v, a_bufs, b_bufs, asems, bsems, osems):
        wid = lax.axis_index("s") * NC + lax.axis_index("c")
        base = wid * tpw
        pltpu.sync_copy(pos1_hbm.at[pl.ds(base, tpw)], p1_v)
        pltpu.sync_copy(pos2_hbm.at[pl.ds(base, tpw)], p2_v)
        inA = [None] * n_chunks
        inB = [None] * n_chunks
        out_d = [None] * n_chunks

        def process(c):
            s = c % nbuf
            inA[c].wait()
            inB[c].wait()
            a_v, b_v = a_bufs[s], b_bufs[s]

            def add_step(j, c2):
                r = j // ncol
                col = (j % ncol) * lanes
                a_v[r, pl.ds(col, lanes)] = (
                    a_v[r, pl.ds(col, lanes)] + b_v[r, pl.ds(col, lanes)]
                )
                return c2

            lax.fori_loop(0, chunk * ncol, add_step, 0, unroll=8)
            out_d[c] = pltpu.async_copy(
                a_v, out_hbm.at[pl.ds(base + c * chunk, chunk)], osems[s])

        for c in range(n_chunks):
            s = c % nbuf
            if c >= nbuf:
                out_d[c - nbuf].wait()
            inA[c] = pltpu.async_copy(
                yg_hbm.at[p1_v.at[pl.ds(c * chunk, chunk)]], a_bufs[s],
                asems[s])
            inB[c] = pltpu.async_copy(
                yg_hbm.at[p2_v.at[pl.ds(c * chunk, chunk)]], b_bufs[s],
                bsems[s])
            if c >= 1:
                process(c - 1)
        process(n_chunks - 1)
        for c in range(max(0, n_chunks - nbuf), n_chunks):
            out_d[c].wait()

    return k(yg, pos1, pos2)


def _tc_expert_matmul(xg, w1, w2, gates3, expert_map):
    def body(em_ref, xg_ref, w1_ref, w2_ref, g_ref, yg_ref):
        xb = xg_ref[...].astype(jnp.bfloat16)
        h = lax.dot_general(
            xb, w1_ref[0].astype(jnp.bfloat16),
            (((1,), (1,)), ((), ())),
            preferred_element_type=jnp.float32,
        )
        h = h * jax.nn.sigmoid(h)
        y = lax.dot_general(
            h.astype(jnp.bfloat16),
            w2_ref[0].astype(jnp.bfloat16),
            (((1,), (1,)), ((), ())),
            preferred_element_type=jnp.float32,
        )
        gv = g_ref[0, 0, :]
        yg_ref[...] = y * gv[:, None]

    grid_spec = pltpu.PrefetchScalarGridSpec(
        num_scalar_prefetch=1,
        grid=(NT,),
        in_specs=[
            pl.BlockSpec((TM, HIDDEN), lambda i, em: (i, 0)),
            pl.BlockSpec((1, INTER, HIDDEN), lambda i, em: (em[i], 0, 0)),
            pl.BlockSpec((1, HIDDEN, INTER), lambda i, em: (em[i], 0, 0)),
            pl.BlockSpec((1, 1, TM), lambda i, em: (i, 0, 0)),
        ],
        out_specs=pl.BlockSpec((TM, HIDDEN), lambda i, em: (i, 0)),
    )
    return pl.pallas_call(
        body,
        grid_spec=grid_spec,
        out_shape=jax.ShapeDtypeStruct((P, HIDDEN), jnp.float32),
        compiler_params=pltpu.CompilerParams(
            dimension_semantics=("arbitrary",),
        ),
    )(expert_map, xg, w1, w2, gates3)


def kernel(x, router_logits, w1, w2):
    lg = router_logits.astype(jnp.float32)
    i1 = jnp.argmax(lg, axis=-1)
    eiota = jnp.arange(N_EXPERTS, dtype=jnp.int32)[None, :]
    oh1 = eiota == i1[:, None]
    lgm = jnp.where(oh1, -jnp.inf, lg)
    i2 = jnp.argmax(lgm, axis=-1)
    oh2 = eiota == i2[:, None]
    l1 = jnp.sum(jnp.where(oh1, lg, 0.0), axis=-1)
    l2 = jnp.sum(jnp.where(oh2, lg, 0.0), axis=-1)
    e2 = jnp.exp(l2 - l1)
    g1 = 1.0 / (1.0 + e2)
    g2 = e2 / (1.0 + e2)

    sel = oh1.astype(jnp.int32) + oh2.astype(jnp.int32)
    csum = jnp.cumsum(sel, axis=0)
    rank = csum - sel
    counts = csum[-1]
    padded = ((counts + TM - 1) // TM) * TM
    starts = jnp.concatenate(
        [jnp.zeros((1,), jnp.int32), jnp.cumsum(padded)[:-1].astype(jnp.int32)]
    )
    pos1 = jnp.sum(jnp.where(oh1, rank + starts[None, :], 0),
                   axis=-1).astype(jnp.int32)
    pos2 = jnp.sum(jnp.where(oh2, rank + starts[None, :], 0),
                   axis=-1).astype(jnp.int32)
    tile_starts = jnp.arange(NT, dtype=jnp.int32) * TM
    expert_map = (tile_starts[:, None] >= starts[None, 1:]).sum(
        axis=1).astype(jnp.int32)

    disp_chunk = 32
    nch = (T // NW) // disp_chunk
    pos1r = pos1.reshape(NW * nch, disp_chunk)
    pos2r = pos2.reshape(NW * nch, disp_chunk)
    xg, gatep = _sc_dispatch(x, pos1r, pos2r, g1, g2,
                             chunk=disp_chunk)

    gates3 = gatep.reshape(NT, 1, TM)
    yg = _tc_expert_matmul(xg, w1, w2, gates3, expert_map)

    out = _sc_combine(yg, pos1, pos2, chunk=16)
    return out.astype(x.dtype)
